# Initial kernel scaffold; baseline (speedup 1.0000x reference)
import functools

import jax
import jax.numpy as jnp
from jax import lax
from jax.experimental import pallas as pl
from jax.experimental.pallas import tpu as pltpu
from jax.experimental.pallas import tpu_sc as plsc

N = 10000
E = 320000
D = 128
NC = 2
NS = 16
NW = NC * NS
EW = E // NW
CH = 80
NCH = EW // CH
RB = 624
BN = 500


def _sc_mesh():
    return plsc.VectorSubcoreMesh(
        core_axis_name="c", subcore_axis_name="s", num_cores=NC, num_subcores=NS
    )


def _deg_call(dst2d, zeros2d):
    @functools.partial(
        pl.kernel,
        out_type=(
            jax.ShapeDtypeStruct((N,), jnp.float32),
            jax.ShapeDtypeStruct((N,), jnp.float32),
        ),
        mesh=_sc_mesh(),
        scratch_types=[
            pltpu.VMEM((NCH, CH), jnp.int32),
            pltpu.VMEM((CH,), jnp.float32),
            pltpu.VMEM_SHARED((N,), jnp.float32),
        ],
    )
    def k(dst_hbm, zero_hbm, out_a, out_b, idx_d, ones_v, acc):
        cid = lax.axis_index("c")
        sid = lax.axis_index("s")
        wid = cid * NS + sid
        pltpu.sync_copy(zero_hbm.at[0, pl.ds(0, RB)], acc.at[pl.ds(sid * RB, RB)])

        @pl.when(sid == NS - 1)
        def _():
            pltpu.sync_copy(zero_hbm.at[0, pl.ds(0, 16)], acc.at[pl.ds(NS * RB, 16)])

        for i in range(CH // 16):
            ones_v[pl.ds(i * 16, 16)] = jnp.ones((16,), jnp.float32)
        pltpu.sync_copy(dst_hbm.at[pl.ds(wid * NCH, NCH)], idx_d)
        plsc.subcore_barrier()

        def body(j, carry):
            pltpu.sync_copy(ones_v, acc.at[idx_d.at[j]], add=True)
            return carry

        lax.fori_loop(0, NCH, body, 0)
        plsc.subcore_barrier()

        @pl.when(cid == 0)
        def _():
            pltpu.sync_copy(acc.at[pl.ds(sid * RB, RB)], out_a.at[pl.ds(sid * RB, RB)])

            @pl.when(sid == NS - 1)
            def _():
                pltpu.sync_copy(acc.at[pl.ds(NS * RB, 16)], out_a.at[pl.ds(NS * RB, 16)])

        @pl.when(cid == 1)
        def _():
            pltpu.sync_copy(acc.at[pl.ds(sid * RB, RB)], out_b.at[pl.ds(sid * RB, RB)])

            @pl.when(sid == NS - 1)
            def _():
                pltpu.sync_copy(acc.at[pl.ds(NS * RB, 16)], out_b.at[pl.ds(NS * RB, 16)])

    return k(dst2d, zeros2d)


def _agg_call(hs, src2d, dst2d, zeros2d):
    @functools.partial(
        pl.kernel,
        out_type=(
            jax.ShapeDtypeStruct((N, D), jnp.float32),
            jax.ShapeDtypeStruct((N, D), jnp.float32),
        ),
        mesh=_sc_mesh(),
        scratch_types=[
            pltpu.VMEM((NCH, CH), jnp.int32),
            pltpu.VMEM((NCH, CH), jnp.int32),
            pltpu.VMEM((CH, D), jnp.float32),
            pltpu.VMEM_SHARED((N, D), jnp.float32),
        ],
    )
    def k(hs_hbm, src_hbm, dst_hbm, zero_hbm, out_a, out_b, idx_s, idx_d, rows, acc):
        cid = lax.axis_index("c")
        sid = lax.axis_index("s")
        wid = cid * NS + sid
        pltpu.sync_copy(zero_hbm.at[pl.ds(0, RB)], acc.at[pl.ds(sid * RB, RB)])

        @pl.when(sid == NS - 1)
        def _():
            pltpu.sync_copy(zero_hbm.at[pl.ds(0, 16)], acc.at[pl.ds(NS * RB, 16)])

        pltpu.sync_copy(src_hbm.at[pl.ds(wid * NCH, NCH)], idx_s)
        pltpu.sync_copy(dst_hbm.at[pl.ds(wid * NCH, NCH)], idx_d)
        plsc.subcore_barrier()

        def body(j, carry):
            pltpu.sync_copy(hs_hbm.at[idx_s.at[j]], rows)
            pltpu.sync_copy(rows, acc.at[idx_d.at[j]], add=True)
            return carry

        lax.fori_loop(0, NCH, body, 0)
        plsc.subcore_barrier()

        @pl.when(cid == 0)
        def _():
            pltpu.sync_copy(acc.at[pl.ds(sid * RB, RB)], out_a.at[pl.ds(sid * RB, RB)])

            @pl.when(sid == NS - 1)
            def _():
                pltpu.sync_copy(acc.at[pl.ds(NS * RB, 16)], out_a.at[pl.ds(NS * RB, 16)])

        @pl.when(cid == 1)
        def _():
            pltpu.sync_copy(acc.at[pl.ds(sid * RB, RB)], out_b.at[pl.ds(sid * RB, RB)])

            @pl.when(sid == NS - 1)
            def _():
                pltpu.sync_copy(acc.at[pl.ds(NS * RB, 16)], out_b.at[pl.ds(NS * RB, 16)])

    return k(hs, src2d, dst2d, zeros2d)


def _dis(deg_ref):
    deg = deg_ref[:, 0:1] + deg_ref[:, 1:2] + 1.0
    return lax.rsqrt(deg)


def _mm1_body(x_ref, w_ref, deg_ref, hs_ref):
    h = jnp.dot(x_ref[...], w_ref[...], preferred_element_type=jnp.float32)
    hs_ref[...] = h * _dis(deg_ref)


def _mm2_body(aa_ref, ab_ref, hs_ref, deg_ref, b_ref, w_ref, out_ref):
    dis = _dis(deg_ref)
    z = dis * (aa_ref[...] + ab_ref[...] + hs_ref[...]) + b_ref[...]
    h = jnp.maximum(z, 0.0)
    out_ref[...] = jnp.dot(h, w_ref[...], preferred_element_type=jnp.float32) * dis


def _fin_body(aa_ref, ab_ref, hs_ref, deg_ref, b_ref, out_ref):
    dis = _dis(deg_ref)
    out_ref[...] = dis * (aa_ref[...] + ab_ref[...] + hs_ref[...]) + b_ref[...]


_row = pl.BlockSpec((500, D), lambda i: (i, 0))
_deg_spec = pl.BlockSpec((500, 2), lambda i: (i, 0))
_full = pl.BlockSpec((D, D), lambda i: (0, 0))
_bias = pl.BlockSpec((1, D), lambda i: (0, 0))
_G = N // 500


def _mm1_call(x, W1, degt):
    return pl.pallas_call(
        _mm1_body,
        grid=(_G,),
        in_specs=[_row, _full, _deg_spec],
        out_specs=_row,
        out_shape=jax.ShapeDtypeStruct((N, D), jnp.float32),
    )(x, W1, degt)


def _mm2_call(agg_a, agg_b, hs1, degt, b1, W2):
    return pl.pallas_call(
        _mm2_body,
        grid=(_G,),
        in_specs=[_row, _row, _row, _deg_spec, _bias, _full],
        out_specs=_row,
        out_shape=jax.ShapeDtypeStruct((N, D), jnp.float32),
    )(agg_a, agg_b, hs1, degt, b1, W2)


def _fin_call(agg_a, agg_b, hs2, degt, b2):
    return pl.pallas_call(
        _fin_body,
        grid=(_G,),
        in_specs=[_row, _row, _row, _deg_spec, _bias],
        out_specs=_row,
        out_shape=jax.ShapeDtypeStruct((N, D), jnp.float32),
    )(agg_a, agg_b, hs2, degt, b2)


def kernel(x, edge_index, W1, b1, W2, b2):
    src = edge_index[0].astype(jnp.int32).reshape(NW * NCH, CH)
    dst = edge_index[1].astype(jnp.int32).reshape(NW * NCH, CH)
    zeros2d = jnp.zeros((RB + 16, D), jnp.float32)

    deg_a, deg_b = _deg_call(dst, zeros2d)
    degt = jnp.stack([deg_a, deg_b], axis=1)

    hs1 = _mm1_call(x, W1, degt)
    agg1a, agg1b = _agg_call(hs1, src, dst, zeros2d)
    hs2 = _mm2_call(agg1a, agg1b, hs1, degt, b1.reshape(1, D), W2)
    agg2a, agg2b = _agg_call(hs2, src, dst, zeros2d)
    return _fin_call(agg2a, agg2b, hs2, degt, b2.reshape(1, D))


# speedup vs baseline: 14.5087x; 14.5087x over previous
import functools

import jax
import jax.numpy as jnp
from jax import lax
from jax.experimental import pallas as pl
from jax.experimental.pallas import tpu as pltpu
from jax.experimental.pallas import tpu_sc as plsc

N = 10000
E = 320000
D = 128
H = D // 2
NC = 2
NS = 16
NW = NC * NS
CH = 80
NCH = E // NS // CH
NCHD = E // NW // CH
RB = 624
ZB = 104
BN = 1000


def _sc_mesh():
    return plsc.VectorSubcoreMesh(
        core_axis_name="c", subcore_axis_name="s", num_cores=NC, num_subcores=NS
    )


def _deg_call(dst2d, zeros1d, ones1d):
    @functools.partial(
        pl.kernel,
        out_type=(
            jax.ShapeDtypeStruct((N,), jnp.float32),
            jax.ShapeDtypeStruct((N,), jnp.float32),
        ),
        mesh=_sc_mesh(),
        scratch_types=[
            pltpu.VMEM((NCHD, CH), jnp.int32),
            pltpu.VMEM((CH,), jnp.float32),
            pltpu.VMEM((RB + 16,), jnp.float32),
            pltpu.VMEM_SHARED((N,), jnp.float32),
        ],
    )
    def k(dst_hbm, zero_hbm, ones_hbm, out_a, out_b, idx_d, ones_v, zbuf, acc):
        cid = lax.axis_index("c")
        sid = lax.axis_index("s")
        wid = cid * NS + sid
        pltpu.sync_copy(zero_hbm, zbuf)
        pltpu.sync_copy(zbuf.at[pl.ds(0, RB)], acc.at[pl.ds(sid * RB, RB)])

        @pl.when(sid == NS - 1)
        def _():
            pltpu.sync_copy(zbuf.at[pl.ds(0, 16)], acc.at[pl.ds(NS * RB, 16)])

        pltpu.sync_copy(ones_hbm, ones_v)
        pltpu.sync_copy(dst_hbm.at[wid], idx_d)
        plsc.subcore_barrier()

        def body(j, carry):
            pltpu.sync_copy(ones_v, acc.at[idx_d.at[j]], add=True)
            return carry

        lax.fori_loop(0, NCHD, body, 0)
        plsc.subcore_barrier()

        pltpu.sync_copy(acc.at[pl.ds(sid * RB, RB)], zbuf.at[pl.ds(0, RB)])
        out = [out_a, out_b]
        for c in range(NC):

            @pl.when(cid == c)
            def _(c=c):
                pltpu.sync_copy(zbuf.at[pl.ds(0, RB)], out[c].at[pl.ds(sid * RB, RB)])

                @pl.when(sid == NS - 1)
                def _():
                    pltpu.sync_copy(acc.at[pl.ds(NS * RB, 16)], zbuf.at[pl.ds(RB, 16)])
                    pltpu.sync_copy(zbuf.at[pl.ds(RB, 16)], out[c].at[pl.ds(NS * RB, 16)])

    return k(dst2d, zeros1d, ones1d)


def _agg_call(hs_a, hs_b, src3, dst3, zeros2d):
    @functools.partial(
        pl.kernel,
        out_type=(
            jax.ShapeDtypeStruct((N, H), jnp.float32),
            jax.ShapeDtypeStruct((N, H), jnp.float32),
        ),
        mesh=_sc_mesh(),
        compiler_params=pltpu.CompilerParams(use_tc_tiling_on_sc=False),
        scratch_types=[
            pltpu.VMEM((NCH, CH), jnp.int32),
            pltpu.VMEM((NCH, CH), jnp.int32),
            pltpu.VMEM((CH, H), jnp.float32),
            pltpu.VMEM((ZB, H), jnp.float32),
            pltpu.VMEM_SHARED((N, H), jnp.float32),
        ],
    )
    def k(hsa_hbm, hsb_hbm, src_hbm, dst_hbm, zero_hbm, out_a, out_b,
          idx_s, idx_d, rows, zbuf, acc):
        cid = lax.axis_index("c")
        sid = lax.axis_index("s")
        pltpu.sync_copy(zero_hbm, zbuf)
        for kk in range(RB // ZB):
            pltpu.sync_copy(zbuf, acc.at[pl.ds(sid * RB + kk * ZB, ZB)])

        @pl.when(sid == NS - 1)
        def _():
            pltpu.sync_copy(zbuf.at[pl.ds(0, 16)], acc.at[pl.ds(NS * RB, 16)])

        pltpu.sync_copy(src_hbm.at[sid], idx_s)
        pltpu.sync_copy(dst_hbm.at[sid], idx_d)
        plsc.subcore_barrier()

        hsp = [hsa_hbm, hsb_hbm]
        for c in range(NC):

            @pl.when(cid == c)
            def _(c=c):
                def body(j, carry):
                    pltpu.sync_copy(hsp[c].at[idx_s.at[j]], rows)
                    pltpu.sync_copy(rows, acc.at[idx_d.at[j]], add=True)
                    return carry

                lax.fori_loop(0, NCH, body, 0)

        plsc.subcore_barrier()

        out = [out_a, out_b]
        for c in range(NC):

            @pl.when(cid == c)
            def _(c=c):
                for kk in range(RB // ZB):
                    pltpu.sync_copy(acc.at[pl.ds(sid * RB + kk * ZB, ZB)], zbuf)
                    pltpu.sync_copy(zbuf, out[c].at[pl.ds(sid * RB + kk * ZB, ZB)])

                @pl.when(sid == NS - 1)
                def _():
                    pltpu.sync_copy(acc.at[pl.ds(NS * RB, 16)], rows.at[pl.ds(0, 16)])
                    pltpu.sync_copy(rows.at[pl.ds(0, 16)], out[c].at[pl.ds(NS * RB, 16)])

    return k(hs_a, hs_b, src3, dst3, zeros2d)


def _dis(deg_ref):
    deg = deg_ref[:, 0:1] + deg_ref[:, 1:2] + 1.0
    return lax.rsqrt(deg)


def _mm1_body(x_ref, w_ref, deg_ref, hsa_ref, hsb_ref):
    h = jnp.dot(x_ref[...], w_ref[...], preferred_element_type=jnp.float32)
    hs = h * _dis(deg_ref)
    hsa_ref[...] = hs[:, :H]
    hsb_ref[...] = hs[:, H:]


def _mm2_body(aa_ref, ab_ref, hsa_ref, hsb_ref, deg_ref, b_ref, w_ref,
              oa_ref, ob_ref):
    dis = _dis(deg_ref)
    za = dis * (aa_ref[...] + hsa_ref[...]) + b_ref[:, :H]
    zb = dis * (ab_ref[...] + hsb_ref[...]) + b_ref[:, H:]
    h = jnp.maximum(jnp.concatenate([za, zb], axis=1), 0.0)
    hs2 = jnp.dot(h, w_ref[...], preferred_element_type=jnp.float32) * dis
    oa_ref[...] = hs2[:, :H]
    ob_ref[...] = hs2[:, H:]


def _fin_body(aa_ref, ab_ref, hsa_ref, hsb_ref, deg_ref, b_ref, out_ref):
    dis = _dis(deg_ref)
    za = dis * (aa_ref[...] + hsa_ref[...]) + b_ref[:, :H]
    zb = dis * (ab_ref[...] + hsb_ref[...]) + b_ref[:, H:]
    out_ref[...] = jnp.concatenate([za, zb], axis=1)


_row = pl.BlockSpec((BN, D), lambda i: (i, 0))
_half = pl.BlockSpec((BN, H), lambda i: (i, 0))
_deg_spec = pl.BlockSpec((BN, 2), lambda i: (i, 0))
_full = pl.BlockSpec((D, D), lambda i: (0, 0))
_bias = pl.BlockSpec((1, D), lambda i: (0, 0))
_G = N // BN
_half_out = jax.ShapeDtypeStruct((N, H), jnp.float32)


def _mm1_call(x, W1, degt):
    return pl.pallas_call(
        _mm1_body,
        grid=(_G,),
        in_specs=[_row, _full, _deg_spec],
        out_specs=(_half, _half),
        out_shape=(_half_out, _half_out),
    )(x, W1, degt)


def _mm2_call(agg_a, agg_b, hs1a, hs1b, degt, b1, W2):
    return pl.pallas_call(
        _mm2_body,
        grid=(_G,),
        in_specs=[_half, _half, _half, _half, _deg_spec, _bias, _full],
        out_specs=(_half, _half),
        out_shape=(_half_out, _half_out),
    )(agg_a, agg_b, hs1a, hs1b, degt, b1, W2)


def _fin_call(agg_a, agg_b, hs2a, hs2b, degt, b2):
    return pl.pallas_call(
        _fin_body,
        grid=(_G,),
        in_specs=[_half, _half, _half, _half, _deg_spec, _bias],
        out_specs=_row,
        out_shape=jax.ShapeDtypeStruct((N, D), jnp.float32),
    )(agg_a, agg_b, hs2a, hs2b, degt, b2)


def kernel(x, edge_index, W1, b1, W2, b2):
    src_d = edge_index[0].astype(jnp.int32).reshape(NW, NCHD, CH)
    dst_d = edge_index[1].astype(jnp.int32).reshape(NW, NCHD, CH)
    src_s = edge_index[0].astype(jnp.int32).reshape(NS, NCH, CH)
    dst_s = edge_index[1].astype(jnp.int32).reshape(NS, NCH, CH)
    zeros2d = jnp.zeros((ZB, H), jnp.float32)
    zeros1d = jnp.zeros((RB + 16,), jnp.float32)
    ones1d = jnp.ones((CH,), jnp.float32)

    deg_a, deg_b = _deg_call(dst_d, zeros1d, ones1d)
    degt = jnp.stack([deg_a, deg_b], axis=1)

    hs1a, hs1b = _mm1_call(x, W1, degt)
    agg1a, agg1b = _agg_call(hs1a, hs1b, src_s, dst_s, zeros2d)
    hs2a, hs2b = _mm2_call(agg1a, agg1b, hs1a, hs1b, degt, b1.reshape(1, D), W2)
    agg2a, agg2b = _agg_call(hs2a, hs2b, src_s, dst_s, zeros2d)
    return _fin_call(agg2a, agg2b, hs2a, hs2b, degt, b2.reshape(1, D))
